# baseline (speedup 1.0000x reference)
import functools

import jax
import jax.numpy as jnp
from jax import lax
from jax.experimental import pallas as pl
from jax.experimental.pallas import tpu as pltpu
from jax.experimental.pallas import tpu_sc as plsc

_N_POS = 4096
_NUM_NEG = 16384
_THRESHOLD = 1.0
_BETA = round(0.2 * _NUM_NEG) / _NUM_NEG
_SCALE = 1.0 / (_N_POS * _NUM_NEG * _BETA)

_BP = 1024
_BN = 8192
_GI = _N_POS // _BP
_GJ = _NUM_NEG // _BN


def _sc_gather_lam(table, idx):
    info = plsc.get_sparse_core_info()
    nw = info.num_cores * info.num_subcores
    b_per_w = _N_POS // nw
    mesh = plsc.VectorSubcoreMesh(core_axis_name="c", subcore_axis_name="s")

    @functools.partial(
        pl.kernel,
        mesh=mesh,
        out_type=jax.ShapeDtypeStruct((_N_POS,), jnp.float32),
        scratch_types=[
            pltpu.VMEM((b_per_w,), jnp.int32),
            pltpu.VMEM((b_per_w,), jnp.float32),
            pltpu.SemaphoreType.DMA,
        ],
    )
    def gather_kernel(table_hbm, idx_hbm, out_hbm, idx_v, rows_v, sem):
        wid = lax.axis_index("s") * info.num_cores + lax.axis_index("c")
        base = wid * b_per_w
        pltpu.sync_copy(idx_hbm.at[pl.ds(base, b_per_w)], idx_v)
        pltpu.async_copy(table_hbm.at[idx_v], rows_v, sem).wait()
        pltpu.sync_copy(rows_v, out_hbm.at[pl.ds(base, b_per_w)])

    return gather_kernel(table, idx)


def _hinge_body(fns_ref, fps_ref, out_ref, acc_ref):
    i = pl.program_id(0)
    j = pl.program_id(1)

    b = fns_ref[...].astype(jnp.bfloat16)
    c = (fps_ref[...] - _THRESHOLD).astype(jnp.bfloat16)

    x = b[None, :] - c
    v = jnp.maximum(x, jnp.bfloat16(0))
    w = v * v
    ones = jnp.ones((_BN, 8), jnp.bfloat16)
    part = jax.lax.dot_general(w, ones,
                               (((1,), (0,)), ((), ())),
                               preferred_element_type=jnp.float32)

    @pl.when(j == 0)
    def _acc_init():
        acc_ref[...] = part

    @pl.when(j > 0)
    def _acc_add():
        acc_ref[...] = acc_ref[...] + part

    @pl.when(j == _GJ - 1)
    def _finish():
        psum = jnp.sum(acc_ref[...]) * 0.125

        @pl.when(i == 0)
        def _init():
            out_ref[0, 0] = 0.0

        out_ref[0, 0] += psum


def _hinge_sum(f_ps, f_ns):
    return pl.pallas_call(
        _hinge_body,
        grid=(_GI, _GJ),
        in_specs=[
            pl.BlockSpec((_BN,), lambda i, j: (j,)),
            pl.BlockSpec((_BP, 1), lambda i, j: (i, 0)),
        ],
        out_specs=pl.BlockSpec((1, 1), lambda i, j: (0, 0),
                               memory_space=pltpu.SMEM),
        out_shape=jax.ShapeDtypeStruct((1, 1), jnp.float32),
        scratch_shapes=[
            pltpu.VMEM((_BP, 8), jnp.float32),
        ],
        compiler_params=pltpu.CompilerParams(
            dimension_semantics=("arbitrary", "arbitrary"),
        ),
    )(f_ns, f_ps)


def _corr_body(fns_ref, fps_ref, lam_ref, tot_ref, out_ref):
    i = pl.program_id(0)
    j = pl.program_id(1)

    lam = lam_ref[...]

    @pl.when((i == 0) & (j == 0))
    def _init():
        out_ref[0, 0] = tot_ref[0, 0]

    @pl.when(jnp.max(lam) > 0.0)
    def _band():
        b = fns_ref[...].astype(jnp.bfloat16)
        c = (fps_ref[...] - _THRESHOLD).astype(jnp.bfloat16)
        s = jnp.sqrt(jnp.maximum(lam, 0.0)).astype(jnp.bfloat16)
        x = b[None, :] - c
        u = jnp.where(x > jnp.bfloat16(0), x, jnp.bfloat16(0))
        v = jnp.where(x <= s, u, jnp.bfloat16(0))
        w = v * v
        ones = jnp.ones((_BN, 8), jnp.bfloat16)
        part = jax.lax.dot_general(w, ones,
                                   (((1,), (0,)), ((), ())),
                                   preferred_element_type=jnp.float32)
        out_ref[0, 0] += -jnp.sum(part) * 0.125

    @pl.when((i == _GI - 1) & (j == _GJ - 1))
    def _scale():
        out_ref[0, 0] = out_ref[0, 0] * _SCALE


def _corrected_loss(f_ps, lam, f_ns, total):
    return pl.pallas_call(
        _corr_body,
        grid=(_GI, _GJ),
        in_specs=[
            pl.BlockSpec((_BN,), lambda i, j: (j,)),
            pl.BlockSpec((_BP, 1), lambda i, j: (i, 0)),
            pl.BlockSpec((_BP, 1), lambda i, j: (i, 0)),
            pl.BlockSpec((1, 1), lambda i, j: (0, 0),
                         memory_space=pltpu.SMEM),
        ],
        out_specs=pl.BlockSpec((1, 1), lambda i, j: (0, 0),
                               memory_space=pltpu.SMEM),
        out_shape=jax.ShapeDtypeStruct((1, 1), jnp.float32),
        compiler_params=pltpu.CompilerParams(
            dimension_semantics=("arbitrary", "arbitrary"),
        ),
    )(f_ns, f_ps, lam, total)


def kernel(y_pred, y_true, index_p, lambda_pos):
    f_ps = y_pred[:_N_POS].reshape(_N_POS, 1)
    f_ns = y_pred[_N_POS:]
    lam = _sc_gather_lam(lambda_pos.reshape(-1), index_p)
    total = _hinge_sum(f_ps, f_ns)
    out = _corrected_loss(f_ps, lam.reshape(_N_POS, 1), f_ns, total)
    return out[0, 0]


# speedup vs baseline: 1.1258x; 1.1258x over previous
import functools

import jax
import jax.numpy as jnp
from jax import lax
from jax.experimental import pallas as pl
from jax.experimental.pallas import tpu as pltpu
from jax.experimental.pallas import tpu_sc as plsc

_N_POS = 4096
_NUM_NEG = 16384
_THRESHOLD = 1.0
_BETA = round(0.2 * _NUM_NEG) / _NUM_NEG
_SCALE = 1.0 / (_N_POS * _NUM_NEG * _BETA)

_BP = 1024
_BN = 8192
_GI = _N_POS // _BP
_GJ = _NUM_NEG // _BN


def _sc_gather_lam(table, idx):
    info = plsc.get_sparse_core_info()
    nw = info.num_cores * info.num_subcores
    b_per_w = _N_POS // nw
    mesh = plsc.VectorSubcoreMesh(core_axis_name="c", subcore_axis_name="s")

    @functools.partial(
        pl.kernel,
        mesh=mesh,
        out_type=jax.ShapeDtypeStruct((_N_POS,), jnp.float32),
        scratch_types=[
            pltpu.VMEM((b_per_w,), jnp.int32),
            pltpu.VMEM((b_per_w,), jnp.float32),
            pltpu.SemaphoreType.DMA,
        ],
    )
    def gather_kernel(table_hbm, idx_hbm, out_hbm, idx_v, rows_v, sem):
        wid = lax.axis_index("s") * info.num_cores + lax.axis_index("c")
        base = wid * b_per_w
        pltpu.sync_copy(idx_hbm.at[pl.ds(base, b_per_w)], idx_v)
        pltpu.async_copy(table_hbm.at[idx_v], rows_v, sem).wait()
        pltpu.sync_copy(rows_v, out_hbm.at[pl.ds(base, b_per_w)])

    return gather_kernel(table, idx)


def _pair_body(fns_ref, fps_ref, lam_ref, out_ref, acc_ref):
    i = pl.program_id(0)
    j = pl.program_id(1)

    b = fns_ref[...].astype(jnp.bfloat16)
    c = (fps_ref[...] - _THRESHOLD).astype(jnp.bfloat16).reshape(_BP, 1)

    x = b[None, :] - c
    v = jnp.maximum(x, jnp.bfloat16(0))
    w = v * v
    ones = jnp.ones((_BN, 8), jnp.bfloat16)
    part = jax.lax.dot_general(w, ones,
                               (((1,), (0,)), ((), ())),
                               preferred_element_type=jnp.float32)

    @pl.when(j == 0)
    def _acc_init():
        acc_ref[...] = part

    @pl.when(j > 0)
    def _acc_add():
        acc_ref[...] = acc_ref[...] + part

    lam = lam_ref[...].reshape(_BP, 1)
    @pl.when(jnp.max(lam) > 0.0)
    def _band():
        s = jnp.sqrt(jnp.maximum(lam, 0.0)).astype(jnp.bfloat16)
        u = jnp.where(x > jnp.bfloat16(0), x, jnp.bfloat16(0))
        g = jnp.where(x <= s, u, jnp.bfloat16(0))
        h = g * g
        band = jax.lax.dot_general(h, ones,
                                   (((1,), (0,)), ((), ())),
                                   preferred_element_type=jnp.float32)
        acc_ref[...] = acc_ref[...] - band

    @pl.when(j == _GJ - 1)
    def _finish():
        psum = jnp.sum(acc_ref[...]) * 0.125

        @pl.when(i == 0)
        def _init():
            out_ref[0, 0] = 0.0

        out_ref[0, 0] += psum

        @pl.when(i == _GI - 1)
        def _scale():
            out_ref[0, 0] = out_ref[0, 0] * _SCALE


def _pair_loss(f_ps, lam, f_ns):
    return pl.pallas_call(
        _pair_body,
        grid=(_GI, _GJ),
        in_specs=[
            pl.BlockSpec((_BN,), lambda i, j: (j,)),
            pl.BlockSpec((_BP,), lambda i, j: (i,)),
            pl.BlockSpec((_BP,), lambda i, j: (i,)),
        ],
        out_specs=pl.BlockSpec((1, 1), lambda i, j: (0, 0),
                               memory_space=pltpu.SMEM),
        out_shape=jax.ShapeDtypeStruct((1, 1), jnp.float32),
        scratch_shapes=[
            pltpu.VMEM((_BP, 8), jnp.float32),
        ],
        compiler_params=pltpu.CompilerParams(
            dimension_semantics=("arbitrary", "arbitrary"),
        ),
    )(f_ns, f_ps, lam)


def kernel(y_pred, y_true, index_p, lambda_pos):
    f_ps = y_pred[:_N_POS]
    f_ns = y_pred[_N_POS:]
    lam = _sc_gather_lam(lambda_pos.reshape(-1), index_p)
    return _pair_loss(f_ps, lam, f_ns)[0, 0]


# speedup vs baseline: 1.1626x; 1.0327x over previous
import functools

import jax
import jax.numpy as jnp
from jax import lax
from jax.experimental import pallas as pl
from jax.experimental.pallas import tpu as pltpu
from jax.experimental.pallas import tpu_sc as plsc

_N_POS = 4096
_NUM_NEG = 16384
_THRESHOLD = 1.0
_BETA = round(0.2 * _NUM_NEG) / _NUM_NEG
_SCALE = 1.0 / (_N_POS * _NUM_NEG * _BETA)

_BP = 512
_BN = 16384
_GI = _N_POS // _BP
_GJ = _NUM_NEG // _BN


def _sc_gather_lam(table, idx):
    info = plsc.get_sparse_core_info()
    nw = info.num_cores * info.num_subcores
    b_per_w = _N_POS // nw
    mesh = plsc.VectorSubcoreMesh(core_axis_name="c", subcore_axis_name="s")

    @functools.partial(
        pl.kernel,
        mesh=mesh,
        out_type=jax.ShapeDtypeStruct((_N_POS,), jnp.float32),
        scratch_types=[
            pltpu.VMEM((b_per_w,), jnp.int32),
            pltpu.VMEM((b_per_w,), jnp.float32),
            pltpu.SemaphoreType.DMA,
        ],
    )
    def gather_kernel(table_hbm, idx_hbm, out_hbm, idx_v, rows_v, sem):
        wid = lax.axis_index("s") * info.num_cores + lax.axis_index("c")
        base = wid * b_per_w
        pltpu.sync_copy(idx_hbm.at[pl.ds(base, b_per_w)], idx_v)
        pltpu.async_copy(table_hbm.at[idx_v], rows_v, sem).wait()
        pltpu.sync_copy(rows_v, out_hbm.at[pl.ds(base, b_per_w)])

    return gather_kernel(table, idx)


def _pair_body(fns_ref, fps_ref, lam_ref, out_ref, acc_ref):
    i = pl.program_id(0)
    j = pl.program_id(1)

    b = fns_ref[...].astype(jnp.bfloat16)
    c = (fps_ref[...] - _THRESHOLD).astype(jnp.bfloat16).reshape(_BP, 1)

    x = b[None, :] - c
    v = jnp.maximum(x, jnp.bfloat16(0))
    w = v * v
    ones = jnp.ones((_BN, 8), jnp.bfloat16)
    part = jax.lax.dot_general(w, ones,
                               (((1,), (0,)), ((), ())),
                               preferred_element_type=jnp.float32)

    @pl.when(j == 0)
    def _acc_init():
        acc_ref[...] = part

    @pl.when(j > 0)
    def _acc_add():
        acc_ref[...] = acc_ref[...] + part

    lam = lam_ref[...].reshape(_BP, 1)
    @pl.when(jnp.max(lam) > 0.0)
    def _band():
        s = jnp.sqrt(jnp.maximum(lam, 0.0)).astype(jnp.bfloat16)
        u = jnp.where(x > jnp.bfloat16(0), x, jnp.bfloat16(0))
        g = jnp.where(x <= s, u, jnp.bfloat16(0))
        h = g * g
        band = jax.lax.dot_general(h, ones,
                                   (((1,), (0,)), ((), ())),
                                   preferred_element_type=jnp.float32)
        acc_ref[...] = acc_ref[...] - band

    @pl.when(j == _GJ - 1)
    def _finish():
        psum = jnp.sum(acc_ref[...]) * 0.125

        @pl.when(i == 0)
        def _init():
            out_ref[0, 0] = 0.0

        out_ref[0, 0] += psum

        @pl.when(i == _GI - 1)
        def _scale():
            out_ref[0, 0] = out_ref[0, 0] * _SCALE


def _pair_loss(f_ps, lam, f_ns):
    return pl.pallas_call(
        _pair_body,
        grid=(_GI, _GJ),
        in_specs=[
            pl.BlockSpec((_BN,), lambda i, j: (j,)),
            pl.BlockSpec((_BP,), lambda i, j: (i,)),
            pl.BlockSpec((_BP,), lambda i, j: (i,)),
        ],
        out_specs=pl.BlockSpec((1, 1), lambda i, j: (0, 0),
                               memory_space=pltpu.SMEM),
        out_shape=jax.ShapeDtypeStruct((1, 1), jnp.float32),
        scratch_shapes=[
            pltpu.VMEM((_BP, 8), jnp.float32),
        ],
        compiler_params=pltpu.CompilerParams(
            dimension_semantics=("arbitrary", "arbitrary"),
        ),
    )(f_ns, f_ps, lam)


def kernel(y_pred, y_true, index_p, lambda_pos):
    f_ps = y_pred[:_N_POS]
    f_ns = y_pred[_N_POS:]
    lam = _sc_gather_lam(lambda_pos.reshape(-1), index_p)
    return _pair_loss(f_ps, lam, f_ns)[0, 0]


# speedup vs baseline: 1.5710x; 1.3513x over previous
import functools

import jax
import jax.numpy as jnp
from jax import lax
from jax.experimental import pallas as pl
from jax.experimental.pallas import tpu as pltpu
from jax.experimental.pallas import tpu_sc as plsc

_N_POS = 4096
_NUM_NEG = 16384
_THRESHOLD = 1.0
_BETA = round(0.2 * _NUM_NEG) / _NUM_NEG
_SCALE = 1.0 / (_N_POS * _NUM_NEG * _BETA)

_BP = 512
_BN = 16384
_GI = _N_POS // _BP
_GJ = _NUM_NEG // _BN


def _sc_gather_lam(table, idx):
    info = plsc.get_sparse_core_info()
    nw = info.num_cores * info.num_subcores
    b_per_w = _N_POS // nw
    mesh = plsc.VectorSubcoreMesh(core_axis_name="c", subcore_axis_name="s")

    @functools.partial(
        pl.kernel,
        mesh=mesh,
        out_type=jax.ShapeDtypeStruct((_N_POS,), jnp.float32),
        scratch_types=[
            pltpu.VMEM((b_per_w,), jnp.int32),
            pltpu.VMEM((b_per_w,), jnp.float32),
            pltpu.SemaphoreType.DMA,
        ],
    )
    def gather_kernel(table_hbm, idx_hbm, out_hbm, idx_v, rows_v, sem):
        wid = lax.axis_index("s") * info.num_cores + lax.axis_index("c")
        base = wid * b_per_w
        pltpu.sync_copy(idx_hbm.at[pl.ds(base, b_per_w)], idx_v)
        pltpu.async_copy(table_hbm.at[idx_v], rows_v, sem).wait()
        pltpu.sync_copy(rows_v, out_hbm.at[pl.ds(base, b_per_w)])

    return gather_kernel(table, idx)


def _pair_body(fns_ref, fps_ref, lam_ref, out_ref, acc_ref):
    i = pl.program_id(0)
    j = pl.program_id(1)

    b = fns_ref[...].astype(jnp.bfloat16)
    c = (fps_ref[...] - _THRESHOLD).astype(jnp.bfloat16).reshape(_BP, 1)

    x = b[None, :] - c
    v = jnp.maximum(x, jnp.bfloat16(0))
    w = v * v

    def _tree(z):
        for _ in range(5):
            h = z.shape[1] // 2
            z = z[:, :h] + z[:, h:]
        return z

    ones = jnp.ones((_BN // 32, 8), jnp.bfloat16)
    part = jax.lax.dot_general(_tree(w), ones,
                               (((1,), (0,)), ((), ())),
                               preferred_element_type=jnp.float32)

    @pl.when(j == 0)
    def _acc_init():
        acc_ref[...] = part

    @pl.when(j > 0)
    def _acc_add():
        acc_ref[...] = acc_ref[...] + part

    lam = lam_ref[...].reshape(_BP, 1)
    @pl.when(jnp.max(lam) > 0.0)
    def _band():
        s = jnp.sqrt(jnp.maximum(lam, 0.0)).astype(jnp.bfloat16)
        u = jnp.where(x > jnp.bfloat16(0), x, jnp.bfloat16(0))
        g = jnp.where(x <= s, u, jnp.bfloat16(0))
        h = g * g
        band = jax.lax.dot_general(_tree(h), ones,
                                   (((1,), (0,)), ((), ())),
                                   preferred_element_type=jnp.float32)
        acc_ref[...] = acc_ref[...] - band

    @pl.when(j == _GJ - 1)
    def _finish():
        psum = jnp.sum(acc_ref[...]) * 0.125

        @pl.when(i == 0)
        def _init():
            out_ref[0, 0] = 0.0

        out_ref[0, 0] += psum

        @pl.when(i == _GI - 1)
        def _scale():
            out_ref[0, 0] = out_ref[0, 0] * _SCALE


def _pair_loss(f_ps, lam, f_ns):
    return pl.pallas_call(
        _pair_body,
        grid=(_GI, _GJ),
        in_specs=[
            pl.BlockSpec((_BN,), lambda i, j: (j,)),
            pl.BlockSpec((_BP,), lambda i, j: (i,)),
            pl.BlockSpec((_BP,), lambda i, j: (i,)),
        ],
        out_specs=pl.BlockSpec((1, 1), lambda i, j: (0, 0),
                               memory_space=pltpu.SMEM),
        out_shape=jax.ShapeDtypeStruct((1, 1), jnp.float32),
        scratch_shapes=[
            pltpu.VMEM((_BP, 8), jnp.float32),
        ],
        compiler_params=pltpu.CompilerParams(
            dimension_semantics=("arbitrary", "arbitrary"),
        ),
    )(f_ns, f_ps, lam)


def kernel(y_pred, y_true, index_p, lambda_pos):
    f_ps = y_pred[:_N_POS]
    f_ns = y_pred[_N_POS:]
    lam = _sc_gather_lam(lambda_pos.reshape(-1), index_p)
    return _pair_loss(f_ps, lam, f_ns)[0, 0]


# speedup vs baseline: 1.6162x; 1.0288x over previous
import functools

import jax
import jax.numpy as jnp
from jax import lax
from jax.experimental import pallas as pl
from jax.experimental.pallas import tpu as pltpu
from jax.experimental.pallas import tpu_sc as plsc

_N_POS = 4096
_NUM_NEG = 16384
_THRESHOLD = 1.0
_BETA = round(0.2 * _NUM_NEG) / _NUM_NEG
_SCALE = 1.0 / (_N_POS * _NUM_NEG * _BETA)

_BP = 1024
_BN = 16384
_GI = _N_POS // _BP
_GJ = _NUM_NEG // _BN


def _sc_gather_lam(table, idx):
    info = plsc.get_sparse_core_info()
    nw = info.num_cores * info.num_subcores
    b_per_w = _N_POS // nw
    mesh = plsc.VectorSubcoreMesh(core_axis_name="c", subcore_axis_name="s")

    @functools.partial(
        pl.kernel,
        mesh=mesh,
        out_type=jax.ShapeDtypeStruct((_N_POS,), jnp.float32),
        scratch_types=[
            pltpu.VMEM((b_per_w,), jnp.int32),
            pltpu.VMEM((b_per_w,), jnp.float32),
            pltpu.SemaphoreType.DMA,
        ],
    )
    def gather_kernel(table_hbm, idx_hbm, out_hbm, idx_v, rows_v, sem):
        wid = lax.axis_index("s") * info.num_cores + lax.axis_index("c")
        base = wid * b_per_w
        pltpu.sync_copy(idx_hbm.at[pl.ds(base, b_per_w)], idx_v)
        pltpu.async_copy(table_hbm.at[idx_v], rows_v, sem).wait()
        pltpu.sync_copy(rows_v, out_hbm.at[pl.ds(base, b_per_w)])

    return gather_kernel(table, idx)


def _pair_body(fns_ref, fps_ref, lam_ref, out_ref, acc_ref):
    i = pl.program_id(0)
    j = pl.program_id(1)

    b = fns_ref[...].astype(jnp.bfloat16)
    c = (fps_ref[...] - _THRESHOLD).astype(jnp.bfloat16).reshape(_BP, 1)

    x = b[None, :] - c
    v = jnp.maximum(x, jnp.bfloat16(0))
    w = v * v

    def _tree(z):
        for _ in range(5):
            h = z.shape[1] // 2
            z = z[:, :h] + z[:, h:]
        return z

    ones = jnp.ones((_BN // 32, 8), jnp.bfloat16)
    part = jax.lax.dot_general(_tree(w), ones,
                               (((1,), (0,)), ((), ())),
                               preferred_element_type=jnp.float32)

    @pl.when(j == 0)
    def _acc_init():
        acc_ref[...] = part

    @pl.when(j > 0)
    def _acc_add():
        acc_ref[...] = acc_ref[...] + part

    lam = lam_ref[...].reshape(_BP, 1)
    @pl.when(jnp.max(lam) > 0.0)
    def _band():
        s = jnp.sqrt(jnp.maximum(lam, 0.0)).astype(jnp.bfloat16)
        u = jnp.where(x > jnp.bfloat16(0), x, jnp.bfloat16(0))
        g = jnp.where(x <= s, u, jnp.bfloat16(0))
        h = g * g
        band = jax.lax.dot_general(_tree(h), ones,
                                   (((1,), (0,)), ((), ())),
                                   preferred_element_type=jnp.float32)
        acc_ref[...] = acc_ref[...] - band

    @pl.when(j == _GJ - 1)
    def _finish():
        psum = jnp.sum(acc_ref[...]) * 0.125

        @pl.when(i == 0)
        def _init():
            out_ref[0, 0] = 0.0

        out_ref[0, 0] += psum

        @pl.when(i == _GI - 1)
        def _scale():
            out_ref[0, 0] = out_ref[0, 0] * _SCALE


def _pair_loss(f_ps, lam, f_ns):
    return pl.pallas_call(
        _pair_body,
        grid=(_GI, _GJ),
        in_specs=[
            pl.BlockSpec((_BN,), lambda i, j: (j,)),
            pl.BlockSpec((_BP,), lambda i, j: (i,)),
            pl.BlockSpec((_BP,), lambda i, j: (i,)),
        ],
        out_specs=pl.BlockSpec((1, 1), lambda i, j: (0, 0),
                               memory_space=pltpu.SMEM),
        out_shape=jax.ShapeDtypeStruct((1, 1), jnp.float32),
        scratch_shapes=[
            pltpu.VMEM((_BP, 8), jnp.float32),
        ],
        compiler_params=pltpu.CompilerParams(
            dimension_semantics=("arbitrary", "arbitrary"),
        ),
    )(f_ns, f_ps, lam)


def kernel(y_pred, y_true, index_p, lambda_pos):
    f_ps = y_pred[:_N_POS]
    f_ns = y_pred[_N_POS:]
    lam = _sc_gather_lam(lambda_pos.reshape(-1), index_p)
    return _pair_loss(f_ps, lam, f_ns)[0, 0]
